# Initial kernel scaffold; baseline (speedup 1.0000x reference)
import jax
import jax.numpy as jnp
from jax.experimental import pallas as pl


def kernel(nodes, senders, receivers, node_coords):
    raise NotImplementedError("write your pallas kernel here")


# speedup vs baseline: 1.6550x; 1.6550x over previous
import jax
import jax.numpy as jnp
from jax import lax
from jax.experimental import pallas as pl
from jax.experimental.pallas import tpu as pltpu
from jax.experimental.pallas import tpu_sc as plsc

_BD = (16, 16, 16)
_NB = _BD[0] * _BD[1] * _BD[2]
_N = 10000
_E = 160000
_D = 256
_NP = 10240
_EP = 163840
_NPT = _NP // 16
_EPT = _EP // 16
_NV = _NPT // 16
_EV = _EPT // 16
_RCH = 32
_NCH = _NPT // _RCH
_WROW = _EPT // 16
_NH = 4
_DH = _D // _NH

_OB = 0
_OA = 10240
_OV = 20480
_OS = 30720
_OR = 40976
_OT = 0
_OK = 32768
_OP = 43008
_OY = 53264
_OI = 63520
_ASZ = 63536


def _rsqrt(x):
    i = plsc.bitcast(x, jnp.int32)
    y = plsc.bitcast(jnp.int32(0x5F3759DF) - (i >> 1), jnp.float32)
    for _ in range(4):
        y = y * (1.5 - 0.5 * x * y * y)
    return y


def _body(nodes_hbm, send_hbm, recv_hbm, coords_hbm,
          coarse_hbm, w_hbm, bs_hbm, br_hbm, keys_hbm,
          arena, b_coords, b_stats, b_statsall, b_nodes, b_ones, b_szv,
          b_scale, b_w, b_w2,
          spm_stats, spm_bids, spm_acc, spm_sizes2, spm_w):
    c = lax.axis_index("c")
    s = lax.axis_index("s")
    lanes = lax.iota(jnp.int32, 16)

    for d in range(3):
        pltpu.sync_copy(coords_hbm.at[pl.ds(d * _NP + s * _NPT, _NPT)],
                        b_coords.at[d])

    def _mm(v, carry):
        mns, mxs = carry
        out_mn, out_mx = [], []
        for d in range(3):
            cv = b_coords[d, pl.ds(v * 16, 16)]
            out_mn.append(jnp.minimum(mns[d], cv))
            out_mx.append(jnp.maximum(mxs[d], cv))
        return tuple(out_mn), tuple(out_mx)

    inf = jnp.full((16,), jnp.inf, jnp.float32)
    mns, mxs = lax.fori_loop(0, _NV, _mm,
                             ((inf, inf, inf), (-inf, -inf, -inf)))
    for d in range(3):
        b_stats[d, :] = mns[d]
        b_stats[3 + d, :] = mxs[d]
    pltpu.sync_copy(b_stats, spm_stats.at[s])
    plsc.subcore_barrier()

    pltpu.sync_copy(spm_stats, b_statsall)

    def _red(t, carry):
        mns, mxs = carry
        out_mn, out_mx = [], []
        for d in range(3):
            out_mn.append(jnp.minimum(mns[d], b_statsall[t, d, :]))
            out_mx.append(jnp.maximum(mxs[d], b_statsall[t, 3 + d, :]))
        return tuple(out_mn), tuple(out_mx)

    mns, mxs = lax.fori_loop(0, 16, _red,
                             ((inf, inf, inf), (-inf, -inf, -inf)))
    mnv, cellv = [], []
    for d in range(3):
        mn = jnp.full((16,), lax.reduce_min(mns[d], (0,)), jnp.float32)
        mx = jnp.full((16,), lax.reduce_max(mxs[d], (0,)), jnp.float32)
        mnv.append(mn)
        cellv.append((mx - mn) * (1.0 / 16.0))

    def _bid(v, _):
        gi = []
        for d in range(3):
            cv = b_coords[d, pl.ds(v * 16, 16)]
            g = ((cv - mnv[d]) / cellv[d]).astype(jnp.int32)
            gi.append(jnp.minimum(jnp.maximum(g, 0), 15))
        arena[pl.ds(_OB + s * _NPT + v * 16, 16)] = (
            gi[0] * 256 + gi[1] * 16 + gi[2])
        return 0

    lax.fori_loop(0, _NV, _bid, 0)
    pltpu.sync_copy(arena.at[pl.ds(_OB + s * _NPT, _NPT)],
                    spm_bids.at[pl.ds(s * _NPT, _NPT)])
    plsc.subcore_barrier()
    pltpu.sync_copy(spm_bids, arena.at[pl.ds(_OB, _NP)])

    ebase = s * _EPT
    pltpu.sync_copy(send_hbm.at[pl.ds(ebase, _EPT)],
                    arena.at[pl.ds(_OA, _EPT)])
    pltpu.sync_copy(recv_hbm.at[pl.ds(ebase, _EPT)],
                    arena.at[pl.ds(_OV, _EPT)])

    def _gath(v, _):
        sv = arena[pl.ds(_OA + v * 16, 16)]
        rv = arena[pl.ds(_OV + v * 16, 16)]
        bsv = plsc.load_gather(arena, [sv + _OB])
        brv = plsc.load_gather(arena, [rv + _OB])
        arena[pl.ds(_OS + v * 16, 16)] = bsv
        arena[pl.ds(_OR + v * 16, 16)] = brv
        return 0

    lax.fori_loop(0, _EV, _gath, 0)

    @pl.when(c == 0)
    def _():
        pltpu.sync_copy(arena.at[pl.ds(_OS, _EPT)],
                        bs_hbm.at[pl.ds(ebase, _EPT)])
        pltpu.sync_copy(arena.at[pl.ds(_OR, _EPT)],
                        br_hbm.at[pl.ds(ebase, _EPT)])

    @pl.when(c == 1)
    def _():
        def _key(v, _):
            bsv = arena[pl.ds(_OS + v * 16, 16)]
            brv = arena[pl.ds(_OR + v * 16, 16)]
            arena[pl.ds(_OS + v * 16, 16)] = bsv * _NB + brv
            return 0

        lax.fori_loop(0, _EV, _key, 0)
        pltpu.sync_copy(arena.at[pl.ds(_OS, _EPT)],
                        keys_hbm.at[pl.ds(ebase, _EPT)])

    z16 = jnp.zeros((16,), jnp.float32)

    @pl.when(c == 1)
    def _():
        zi16 = jnp.zeros((16,), jnp.int32)

        def _ztbl(i, _):
            arena[pl.ds(_OT + i * 16, 16)] = zi16
            return 0

        lax.fori_loop(0, 32768 // 16, _ztbl, 0)

        def _zw(i, _):
            r, v = i // (_WROW // 16), i % (_WROW // 16)
            b_w[r, pl.ds(v * 16, 16)] = z16
            return 0

        lax.fori_loop(0, 16 * (_WROW // 16), _zw, 0)
        arena[pl.ds(_OI, 16)] = lanes

    for h in range(_NH):
        @pl.when(c == 0)
        def _(h=h):
            def _znodes(i, _):
                r, v = i // (_DH // 16), i % (_DH // 16)
                b_nodes[r, pl.ds(v * 16, 16)] = z16
                return 0

            lax.fori_loop(0, _RCH * (_DH // 16), _znodes, 0)

            def _zacc(chk, _):
                pltpu.sync_copy(b_nodes,
                                spm_acc.at[pl.ds(s * (_NB // 16) + chk * _RCH,
                                                 _RCH)])
                return 0

            lax.fori_loop(0, _NB // 16 // _RCH, _zacc, 0)
            if h == 0:
                def _zsz(i, _):
                    b_szv[i, :] = z16
                    return 0

                lax.fori_loop(0, _NB // 16, _zsz, 0)
                pltpu.sync_copy(b_szv, spm_sizes2.at[pl.ds(s * (_NB // 16),
                                                           _NB // 16)])

        plsc.subcore_barrier()

        @pl.when(c == 0)
        def _(h=h):
            def _seg(ch, _):
                rb = s * _NPT + ch * _RCH
                pltpu.sync_copy(
                    nodes_hbm.at[pl.ds(rb, _RCH), pl.ds(h * _DH, _DH)],
                    b_nodes)
                idx = arena.at[pl.ds(_OB + rb, _RCH)]
                pltpu.sync_copy(b_nodes, spm_acc.at[idx], add=True)
                if h == 0:
                    def _ones(r, _):
                        val = jnp.where(rb + r < _N, 1.0, 0.0)
                        b_ones[r, :] = jnp.full((16,), val, jnp.float32)
                        return 0

                    lax.fori_loop(0, _RCH, _ones, 0)
                    pltpu.sync_copy(b_ones, spm_sizes2.at[idx], add=True)
                return 0

            lax.fori_loop(0, _NCH, _seg, 0)

        plsc.subcore_barrier()

        @pl.when(c == 0)
        def _(h=h):
            rows = _NB // 16
            if h == 0:
                pltpu.sync_copy(spm_sizes2.at[pl.ds(s * rows, rows)], b_szv)

            def _norm(chk, _):
                rb = s * rows + chk * _RCH
                pltpu.sync_copy(spm_acc.at[pl.ds(rb, _RCH)], b_nodes)
                for g in range(_RCH // 16):
                    ridx = chk * _RCH + g * 16 + lanes
                    sz = plsc.load_gather(b_szv, [ridx, lanes * 0])
                    b_scale[pl.ds(g * 16, 16)] = _rsqrt(sz + 1e-10)

                def _scl(r, _):
                    sc = plsc.load_gather(b_scale,
                                          [jnp.full((16,), r, jnp.int32)])
                    for v in range(_DH // 16):
                        b_nodes[r, pl.ds(v * 16, 16)] = (
                            b_nodes[r, pl.ds(v * 16, 16)] * sc)
                    return 0

                lax.fori_loop(0, _RCH, _scl, 0)
                pltpu.sync_copy(
                    b_nodes,
                    coarse_hbm.at[pl.ds(rb, _RCH), pl.ds(h * _DH, _DH)])
                return 0

            lax.fori_loop(0, rows // _RCH, _norm, 0)

    def _chunk(ch, _):
        @pl.when(c == 1)
        def _():
            pltpu.sync_copy(keys_hbm.at[pl.ds(ch * _EPT, _EPT)],
                            arena.at[pl.ds(_OK, _EPT)])

            @pl.when(s == 0)
            def _():
                pltpu.sync_copy(b_w, spm_w)

        plsc.subcore_barrier()

        @pl.when(c == 1)
        def _():
            def _scan(v, cnt):
                k = arena[pl.ds(_OK + v * 16, 16)]
                own = (k & 15) == s
                bsv = k >> 12
                ns = bsv != (k & (_NB - 1))
                pay = ((k >> 4) << 1) | ns.astype(jnp.int32)
                pos = v * 16 + lanes
                plsc.store_compressed(arena.at[pl.ds(_OP + cnt, 16)], pos,
                                      mask=own)
                plsc.store_compressed(arena.at[pl.ds(_OY + cnt, 16)], pay,
                                      mask=own)
                return cnt + jnp.sum(own.astype(jnp.int32))

            cnt = lax.fori_loop(0, _EV, _scan, jnp.int32(0))
            lane0 = lanes == 0

            def _serial(j, _):
                jv = jnp.full((16,), j, jnp.int32)
                pay = plsc.load_gather(arena, [jv + _OY])
                ns = pay & 1
                loc = pay >> 1
                w = loc >> 5
                bit = jnp.int32(1) << (loc & 31)
                old = plsc.load_gather(arena, [w + _OT])
                isnew = (old & bit) == 0
                take = (ns == 1) & isnew
                plsc.store_scatter(arena, [w + _OT],
                                   old | jnp.where(ns == 1, bit, 0),
                                   mask=lane0)
                pos = plsc.load_gather(arena, [jv + _OP])
                plsc.store_scatter(b_w, [pos // _WROW, pos % _WROW],
                                   jnp.where(take, 1.0, 0.0), mask=lane0)
                return 0

            lax.fori_loop(0, cnt, _serial, 0)
            pltpu.sync_copy(b_w, spm_w.at[arena.at[pl.ds(_OI, 16)]],
                            add=True)

            def _reset(j, _):
                jv = jnp.full((16,), j, jnp.int32)
                pos = plsc.load_gather(arena, [jv + _OP])
                plsc.store_scatter(b_w, [pos // _WROW, pos % _WROW],
                                   jnp.zeros((16,), jnp.float32), mask=lane0)
                return 0

            lax.fori_loop(0, cnt, _reset, 0)

        plsc.subcore_barrier()

        @pl.when((c == 1) & (s == 0))
        def _():
            pltpu.sync_copy(spm_w, b_w2)
            pltpu.sync_copy(b_w2, w_hbm.at[pl.ds(ch * 16, 16)])

        plsc.subcore_barrier()
        return 0

    lax.fori_loop(0, 16, _chunk, 0)


def _run(nodes_pad, send_pad, recv_pad, coords_flat):
    mesh = plsc.VectorSubcoreMesh(core_axis_name="c", subcore_axis_name="s",
                                  num_cores=2, num_subcores=16)
    f = pl.kernel(
        _body,
        out_type=(
            jax.ShapeDtypeStruct((_NB, _D), jnp.float32),
            jax.ShapeDtypeStruct((_EP // _WROW, _WROW), jnp.float32),
            jax.ShapeDtypeStruct((_EP,), jnp.int32),
            jax.ShapeDtypeStruct((_EP,), jnp.int32),
            jax.ShapeDtypeStruct((_EP,), jnp.int32),
        ),
        mesh=mesh,
        compiler_params=pltpu.CompilerParams(use_tc_tiling_on_sc=False,
                                             needs_layout_passes=False),
        scratch_types=[
            pltpu.VMEM((_ASZ,), jnp.int32),
            pltpu.VMEM((3, _NPT), jnp.float32),
            pltpu.VMEM((8, 16), jnp.float32),
            pltpu.VMEM((16, 8, 16), jnp.float32),
            pltpu.VMEM((_RCH, _DH), jnp.float32),
            pltpu.VMEM((_RCH, 16), jnp.float32),
            pltpu.VMEM((_NB // 16, 16), jnp.float32),
            pltpu.VMEM((_RCH,), jnp.float32),
            pltpu.VMEM((16, _WROW), jnp.float32),
            pltpu.VMEM((16, _WROW), jnp.float32),
            pltpu.VMEM_SHARED((16, 8, 16), jnp.float32),
            pltpu.VMEM_SHARED((_NP,), jnp.int32),
            pltpu.VMEM_SHARED((_NB, _DH), jnp.float32),
            pltpu.VMEM_SHARED((_NB, 16), jnp.float32),
            pltpu.VMEM_SHARED((16, _WROW), jnp.float32),
        ],
    )
    return f(nodes_pad, send_pad, recv_pad, coords_flat)


def kernel(nodes, senders, receivers, node_coords):
    nodes_pad = jnp.concatenate(
        [nodes, jnp.zeros((_NP - _N, _D), jnp.float32)], axis=0)
    zpad = jnp.zeros((_EP - _E,), jnp.int32)
    send_pad = jnp.concatenate([senders, zpad])
    recv_pad = jnp.concatenate([receivers, zpad])
    ct = node_coords.T
    coords_flat = jnp.concatenate(
        [ct, jnp.broadcast_to(ct[:, :1], (3, _NP - _N))], axis=1).reshape(-1)
    coarse, w_pad, bs_pad, br_pad, _ = _run(nodes_pad, send_pad, recv_pad,
                                            coords_flat)
    edge_weights = w_pad.reshape(_EP)[:_E].reshape(_E, 1)
    return coarse, edge_weights, bs_pad[:_E], br_pad[:_E]


# speedup vs baseline: 8.8400x; 5.3415x over previous
import jax
import jax.numpy as jnp
from jax import lax
from jax.experimental import pallas as pl
from jax.experimental.pallas import tpu as pltpu
from jax.experimental.pallas import tpu_sc as plsc

_BD = (16, 16, 16)
_NB = _BD[0] * _BD[1] * _BD[2]
_N = 10000
_E = 160000
_D = 256
_NP = 10240
_EP = 163840
_NPT = _NP // 16
_EPT = _EP // 16
_NV = _NPT // 16
_EV = _EPT // 16
_RCH = 32
_NCH = _NPT // _RCH
_WROW = 1024
_WRPC = _EPT // _WROW
_NH = 4
_DH = _D // _NH

_OB = 0
_OA = 10240
_OV = 20480
_OS = 30720
_OR = 40976
_OT = 0
_OK = 32768
_OP = 43008
_OY = 53264
_OI = 63520
_ASZ = 63536


def _rsqrt(x):
    i = plsc.bitcast(x, jnp.int32)
    y = plsc.bitcast(jnp.int32(0x5F3759DF) - (i >> 1), jnp.float32)
    for _ in range(4):
        y = y * (1.5 - 0.5 * x * y * y)
    return y


def _body(nodes_hbm, send_hbm, recv_hbm, coords_hbm,
          coarse_hbm, w_hbm, bs_hbm, br_hbm, keys_hbm,
          arena, b_coords, b_stats, b_statsall, b_nodes, b_ones, b_szv,
          b_scale, b_w, b_w2,
          spm_stats, spm_bids, spm_acc, spm_sizes2, spm_w):
    c = lax.axis_index("c")
    s = lax.axis_index("s")
    lanes = lax.iota(jnp.int32, 16)

    for d in range(3):
        pltpu.sync_copy(coords_hbm.at[pl.ds(d * _NP + s * _NPT, _NPT)],
                        b_coords.at[d])

    def _mm(v, carry):
        mns, mxs = carry
        out_mn, out_mx = [], []
        for d in range(3):
            cv = b_coords[d, pl.ds(v * 16, 16)]
            out_mn.append(jnp.minimum(mns[d], cv))
            out_mx.append(jnp.maximum(mxs[d], cv))
        return tuple(out_mn), tuple(out_mx)

    inf = jnp.full((16,), jnp.inf, jnp.float32)
    mns, mxs = lax.fori_loop(0, _NV, _mm,
                             ((inf, inf, inf), (-inf, -inf, -inf)))
    for d in range(3):
        b_stats[d, :] = mns[d]
        b_stats[3 + d, :] = mxs[d]
    pltpu.sync_copy(b_stats, spm_stats.at[s])
    plsc.subcore_barrier()

    pltpu.sync_copy(spm_stats, b_statsall)

    def _red(t, carry):
        mns, mxs = carry
        out_mn, out_mx = [], []
        for d in range(3):
            out_mn.append(jnp.minimum(mns[d], b_statsall[t, d, :]))
            out_mx.append(jnp.maximum(mxs[d], b_statsall[t, 3 + d, :]))
        return tuple(out_mn), tuple(out_mx)

    mns, mxs = lax.fori_loop(0, 16, _red,
                             ((inf, inf, inf), (-inf, -inf, -inf)))
    mnv, cellv = [], []
    for d in range(3):
        mn = jnp.full((16,), lax.reduce_min(mns[d], (0,)), jnp.float32)
        mx = jnp.full((16,), lax.reduce_max(mxs[d], (0,)), jnp.float32)
        mnv.append(mn)
        cellv.append((mx - mn) * (1.0 / 16.0))

    def _bid(v, _):
        gi = []
        for d in range(3):
            cv = b_coords[d, pl.ds(v * 16, 16)]
            g = ((cv - mnv[d]) / cellv[d]).astype(jnp.int32)
            gi.append(jnp.minimum(jnp.maximum(g, 0), 15))
        arena[pl.ds(_OB + s * _NPT + v * 16, 16)] = (
            gi[0] * 256 + gi[1] * 16 + gi[2])
        return 0

    lax.fori_loop(0, _NV, _bid, 0)
    pltpu.sync_copy(arena.at[pl.ds(_OB + s * _NPT, _NPT)],
                    spm_bids.at[pl.ds(s * _NPT, _NPT)])
    plsc.subcore_barrier()
    pltpu.sync_copy(spm_bids, arena.at[pl.ds(_OB, _NP)])

    ebase = s * _EPT
    pltpu.sync_copy(send_hbm.at[pl.ds(ebase, _EPT)],
                    arena.at[pl.ds(_OA, _EPT)])
    pltpu.sync_copy(recv_hbm.at[pl.ds(ebase, _EPT)],
                    arena.at[pl.ds(_OV, _EPT)])

    def _gath(v, _):
        sv = arena[pl.ds(_OA + v * 16, 16)]
        rv = arena[pl.ds(_OV + v * 16, 16)]
        bsv = plsc.load_gather(arena, [sv + _OB])
        brv = plsc.load_gather(arena, [rv + _OB])
        arena[pl.ds(_OS + v * 16, 16)] = bsv
        arena[pl.ds(_OR + v * 16, 16)] = brv
        return 0

    lax.fori_loop(0, _EV, _gath, 0)

    @pl.when(c == 0)
    def _():
        pltpu.sync_copy(arena.at[pl.ds(_OS, _EPT)],
                        bs_hbm.at[pl.ds(ebase, _EPT)])
        pltpu.sync_copy(arena.at[pl.ds(_OR, _EPT)],
                        br_hbm.at[pl.ds(ebase, _EPT)])

    @pl.when(c == 1)
    def _():
        def _key(v, _):
            bsv = arena[pl.ds(_OS + v * 16, 16)]
            brv = arena[pl.ds(_OR + v * 16, 16)]
            arena[pl.ds(_OS + v * 16, 16)] = bsv * _NB + brv
            return 0

        lax.fori_loop(0, _EV, _key, 0)
        pltpu.sync_copy(arena.at[pl.ds(_OS, _EPT)],
                        keys_hbm.at[pl.ds(ebase, _EPT)])

    z16 = jnp.zeros((16,), jnp.float32)

    @pl.when(c == 1)
    def _():
        zi16 = jnp.zeros((16,), jnp.int32)

        def _ztbl(i, _):
            arena[pl.ds(_OT + i * 16, 16)] = zi16
            return 0

        lax.fori_loop(0, 32768 // 16, _ztbl, 0)

        def _zw(i, _):
            r, v = i // (_WROW // 16), i % (_WROW // 16)
            b_w[r, pl.ds(v * 16, 16)] = z16
            return 0

        lax.fori_loop(0, _WRPC * (_WROW // 16), _zw, 0)
        arena[pl.ds(_OI, 16)] = lanes

    for h in range(_NH):
        @pl.when(c == 0)
        def _(h=h):
            def _znodes(i, _):
                r, v = i // (_DH // 16), i % (_DH // 16)
                b_nodes[r, pl.ds(v * 16, 16)] = z16
                return 0

            lax.fori_loop(0, _RCH * (_DH // 16), _znodes, 0)

            def _zacc(chk, _):
                pltpu.sync_copy(b_nodes,
                                spm_acc.at[pl.ds(s * (_NB // 16) + chk * _RCH,
                                                 _RCH)])
                return 0

            lax.fori_loop(0, _NB // 16 // _RCH, _zacc, 0)
            if h == 0:
                def _zsz(i, _):
                    b_szv[i, :] = z16
                    return 0

                lax.fori_loop(0, _NB // 16, _zsz, 0)
                pltpu.sync_copy(b_szv, spm_sizes2.at[pl.ds(s * (_NB // 16),
                                                           _NB // 16)])

        plsc.subcore_barrier()

        @pl.when(c == 0)
        def _(h=h):
            def _seg(ch, _):
                rb = s * _NPT + ch * _RCH
                pltpu.sync_copy(
                    nodes_hbm.at[pl.ds(rb, _RCH), pl.ds(h * _DH, _DH)],
                    b_nodes)
                idx = arena.at[pl.ds(_OB + rb, _RCH)]
                pltpu.sync_copy(b_nodes, spm_acc.at[idx], add=True)
                if h == 0:
                    def _ones(r, _):
                        val = jnp.where(rb + r < _N, 1.0, 0.0)
                        b_ones[r, :] = jnp.full((16,), val, jnp.float32)
                        return 0

                    lax.fori_loop(0, _RCH, _ones, 0)
                    pltpu.sync_copy(b_ones, spm_sizes2.at[idx], add=True)
                return 0

            lax.fori_loop(0, _NCH, _seg, 0)

        plsc.subcore_barrier()

        @pl.when(c == 0)
        def _(h=h):
            rows = _NB // 16
            if h == 0:
                pltpu.sync_copy(spm_sizes2.at[pl.ds(s * rows, rows)], b_szv)

            def _norm(chk, _):
                rb = s * rows + chk * _RCH
                pltpu.sync_copy(spm_acc.at[pl.ds(rb, _RCH)], b_nodes)
                for g in range(_RCH // 16):
                    ridx = chk * _RCH + g * 16 + lanes
                    sz = plsc.load_gather(b_szv, [ridx, lanes * 0])
                    b_scale[pl.ds(g * 16, 16)] = _rsqrt(sz + 1e-10)

                def _scl(r, _):
                    sc = plsc.load_gather(b_scale,
                                          [jnp.full((16,), r, jnp.int32)])
                    for v in range(_DH // 16):
                        b_nodes[r, pl.ds(v * 16, 16)] = (
                            b_nodes[r, pl.ds(v * 16, 16)] * sc)
                    return 0

                lax.fori_loop(0, _RCH, _scl, 0)
                pltpu.sync_copy(
                    b_nodes,
                    coarse_hbm.at[pl.ds(rb, _RCH), pl.ds(h * _DH, _DH)])
                return 0

            lax.fori_loop(0, rows // _RCH, _norm, 0)

    def _chunk(ch, _):
        @pl.when(c == 1)
        def _():
            pltpu.sync_copy(keys_hbm.at[pl.ds(ch * _EPT, _EPT)],
                            arena.at[pl.ds(_OK, _EPT)])

            @pl.when(s == 0)
            def _():
                pltpu.sync_copy(b_w, spm_w)

        plsc.subcore_barrier()

        @pl.when(c == 1)
        def _():
            def _scan(v, cnt):
                k = arena[pl.ds(_OK + v * 16, 16)]
                own = (k & 15) == s
                pos = v * 16 + lanes
                plsc.store_compressed(arena.at[pl.ds(_OP + cnt, 16)], pos,
                                      mask=own)
                plsc.store_compressed(arena.at[pl.ds(_OY + cnt, 16)], k,
                                      mask=own)
                pc = plsc.all_reduce_population_count(own)
                return cnt + pc[0]

            cnt = lax.fori_loop(0, _EV, _scan, jnp.int32(0))
            lane0 = lanes == 0
            zi16 = jnp.zeros((16,), jnp.int32)
            arena[pl.ds(_OP + cnt, 16)] = zi16
            arena[pl.ds(_OY + cnt, 16)] = zi16
            nv = (cnt + 15) >> 4

            def _dedup(v, _):
                vb = v * 16
                valid = (vb + lanes) < cnt
                k = arena[pl.ds(_OY + vb, 16)]
                pos = arena[pl.ds(_OP + vb, 16)]
                ns = ((k >> 12) != (k & (_NB - 1))) & valid
                loc = k >> 4
                w = loc >> 5
                bit = jnp.int32(1) << (loc & 31)
                runc, _lm = plsc.scan_count(w)
                dupfree = (lax.reduce_max(runc, (0,))
                           == lax.reduce_min(runc, (0,)))

                @pl.when(dupfree)
                def _():
                    old = plsc.load_gather(arena, [w + _OT])
                    isnew = ((old & bit) == 0) & ns
                    neww = old | jnp.where(ns, bit, 0)
                    plsc.store_scatter(arena, [w + _OT], neww, mask=valid)
                    plsc.store_scatter(
                        b_w, [pos >> 10, pos & (_WROW - 1)],
                        jnp.where(isnew, 1.0, 0.0), mask=valid)

                @pl.when(jnp.logical_not(dupfree))
                def _():
                    n = jnp.minimum(cnt - vb, 16)

                    def _serial(j, _):
                        jv = jnp.full((16,), vb + j, jnp.int32)
                        k1 = plsc.load_gather(arena, [jv + _OY])
                        ns1 = (k1 >> 12) != (k1 & (_NB - 1))
                        loc1 = k1 >> 4
                        w1 = loc1 >> 5
                        bit1 = jnp.int32(1) << (loc1 & 31)
                        old = plsc.load_gather(arena, [w1 + _OT])
                        take = ns1 & ((old & bit1) == 0)
                        plsc.store_scatter(arena, [w1 + _OT],
                                           old | jnp.where(ns1, bit1, 0),
                                           mask=lane0)
                        pos1 = plsc.load_gather(arena, [jv + _OP])
                        plsc.store_scatter(
                            b_w, [pos1 >> 10, pos1 & (_WROW - 1)],
                            jnp.where(take, 1.0, 0.0), mask=lane0)
                        return 0

                    lax.fori_loop(0, n, _serial, 0)

                return 0

            lax.fori_loop(0, nv, _dedup, 0)
            pltpu.sync_copy(b_w, spm_w.at[arena.at[pl.ds(_OI, _WRPC)]],
                            add=True)

            def _reset(v, _):
                vb = v * 16
                valid = (vb + lanes) < cnt
                pos = arena[pl.ds(_OP + vb, 16)]
                plsc.store_scatter(b_w, [pos >> 10, pos & (_WROW - 1)],
                                   jnp.zeros((16,), jnp.float32), mask=valid)
                return 0

            lax.fori_loop(0, nv, _reset, 0)

        plsc.subcore_barrier()

        @pl.when((c == 1) & (s == 0))
        def _():
            pltpu.sync_copy(spm_w, b_w2)
            pltpu.sync_copy(b_w2, w_hbm.at[pl.ds(ch * _WRPC, _WRPC)])

        plsc.subcore_barrier()
        return 0

    lax.fori_loop(0, 16, _chunk, 0)


def _run(nodes_pad, send_pad, recv_pad, coords_flat):
    mesh = plsc.VectorSubcoreMesh(core_axis_name="c", subcore_axis_name="s",
                                  num_cores=2, num_subcores=16)
    f = pl.kernel(
        _body,
        out_type=(
            jax.ShapeDtypeStruct((_NB, _D), jnp.float32),
            jax.ShapeDtypeStruct((_EP // _WROW, _WROW), jnp.float32),
            jax.ShapeDtypeStruct((_EP,), jnp.int32),
            jax.ShapeDtypeStruct((_EP,), jnp.int32),
            jax.ShapeDtypeStruct((_EP,), jnp.int32),
        ),
        mesh=mesh,
        compiler_params=pltpu.CompilerParams(use_tc_tiling_on_sc=False,
                                             needs_layout_passes=False),
        scratch_types=[
            pltpu.VMEM((_ASZ,), jnp.int32),
            pltpu.VMEM((3, _NPT), jnp.float32),
            pltpu.VMEM((8, 16), jnp.float32),
            pltpu.VMEM((16, 8, 16), jnp.float32),
            pltpu.VMEM((_RCH, _DH), jnp.float32),
            pltpu.VMEM((_RCH, 16), jnp.float32),
            pltpu.VMEM((_NB // 16, 16), jnp.float32),
            pltpu.VMEM((_RCH,), jnp.float32),
            pltpu.VMEM((_WRPC, _WROW), jnp.float32),
            pltpu.VMEM((_WRPC, _WROW), jnp.float32),
            pltpu.VMEM_SHARED((16, 8, 16), jnp.float32),
            pltpu.VMEM_SHARED((_NP,), jnp.int32),
            pltpu.VMEM_SHARED((_NB, _DH), jnp.float32),
            pltpu.VMEM_SHARED((_NB, 16), jnp.float32),
            pltpu.VMEM_SHARED((_WRPC, _WROW), jnp.float32),
        ],
    )
    return f(nodes_pad, send_pad, recv_pad, coords_flat)


def kernel(nodes, senders, receivers, node_coords):
    nodes_pad = jnp.concatenate(
        [nodes, jnp.zeros((_NP - _N, _D), jnp.float32)], axis=0)
    zpad = jnp.zeros((_EP - _E,), jnp.int32)
    send_pad = jnp.concatenate([senders, zpad])
    recv_pad = jnp.concatenate([receivers, zpad])
    ct = node_coords.T
    coords_flat = jnp.concatenate(
        [ct, jnp.broadcast_to(ct[:, :1], (3, _NP - _N))], axis=1).reshape(-1)
    coarse, w_pad, bs_pad, br_pad, _ = _run(nodes_pad, send_pad, recv_pad,
                                            coords_flat)
    edge_weights = w_pad.reshape(_EP)[:_E].reshape(_E, 1)
    return coarse, edge_weights, bs_pad[:_E], br_pad[:_E]
